# baseline (speedup 1.0000x reference)
import functools

import jax
import jax.numpy as jnp
from jax import lax
from jax.experimental import pallas as pl
from jax.experimental.pallas import tpu as pltpu
from jax.experimental.pallas import tpu_sc as plsc

N = 64
D = 128
L = 16
NCHUNK = N // L
NC = 2
NS = 16
NW = NC * NS
APW = N // NW

_mesh = plsc.VectorSubcoreMesh(
    core_axis_name="c", subcore_axis_name="s", num_cores=NC, num_subcores=NS
)


@functools.partial(
    pl.kernel,
    out_type=jax.ShapeDtypeStruct((NC, L), jnp.float32),
    mesh=_mesh,
    compiler_params=pltpu.CompilerParams(needs_layout_passes=False),
    scratch_types=[
        pltpu.VMEM((D * N,), jnp.float32),
        pltpu.VMEM((N,), jnp.int32),
        pltpu.VMEM((N,), jnp.float32),
        pltpu.VMEM((L,), jnp.int32),
        pltpu.VMEM((N,), jnp.float32),
        pltpu.VMEM((N,), jnp.float32),
        pltpu.VMEM((N,), jnp.float32),
        pltpu.VMEM((L,), jnp.float32),
        pltpu.VMEM((1, 2 * L), jnp.float32),
        pltpu.VMEM((1, 2 * L), jnp.float32),
        pltpu.VMEM((1, L), jnp.float32),
        pltpu.VMEM((1,), jnp.int32),
        pltpu.VMEM_SHARED((1, 2 * L), jnp.float32),
    ],
)
def _triplet_sc(et_hbm, lab_hbm, max_hbm, zidx_hbm, out_hbm,
                et_v, lab_v, labf_v, hist_v, valid_v, r_v, base_v, max_v,
                part_v, red_v, wr_v, zidx_v, shared):
    c = lax.axis_index("c")
    s = lax.axis_index("s")
    wid = s * NC + c

    pltpu.sync_copy(et_hbm, et_v)
    pltpu.sync_copy(lab_hbm, lab_v)
    pltpu.sync_copy(max_hbm, max_v)
    pltpu.sync_copy(zidx_hbm, zidx_v)

    iota = lax.iota(jnp.int32, L)
    zf = jnp.zeros((L,), jnp.float32)
    inv = 1.0 / max_v[...]

    @pl.when(s == 0)
    def _():
        part_v[0, pl.ds(0, L)] = zf
        part_v[0, pl.ds(L, L)] = zf
        pltpu.sync_copy(part_v, shared)
    plsc.subcore_barrier()

    for q in range(NCHUNK):
        lc = lab_v[pl.ds(q * L, L)]
        labf_v[pl.ds(q * L, L)] = lc.astype(jnp.float32)
    h = jnp.zeros((L,), jnp.int32)
    for j in range(N):
        lj = plsc.load_gather(lab_v, [jnp.full((L,), j, jnp.int32)])
        h = h + jnp.where(iota == lj, 1, 0).astype(jnp.int32)
    hist_v[...] = h
    for q in range(NCHUNK):
        lc = lab_v[pl.ds(q * L, L)]
        hc = plsc.load_gather(hist_v, [lc])
        valid_v[pl.ds(q * L, L)] = jnp.where(N - hc >= 2, 1.0, 0.0)

    acc_tot = zf
    cnt_tot = zf

    for k in range(APW):
        a = wid * APW + k

        def p1_body(d, accs, a=a):
            base = d * N
            ea = plsc.load_gather(
                et_v, [jnp.full((L,), base + a, jnp.int32)])
            out = []
            for q in range(NCHUNK):
                col = et_v[pl.ds(base + q * L, L)]
                df = col - ea
                out.append(accs[q] + df * df)
            return tuple(out)

        accs = lax.fori_loop(0, D, p1_body, (zf,) * NCHUNK)
        for q in range(NCHUNK):
            r_v[pl.ds(q * L, L)] = accs[q]

        for q in range(NCHUNK):
            vc = valid_v[pl.ds(q * L, L)]
            pc = iota + q * L
            base_v[pl.ds(q * L, L)] = jnp.where(pc != a, vc, 0.0)

        rp_c = [r_v[pl.ds(q * L, L)] for q in range(NCHUNK)]
        lpf_c = [labf_v[pl.ds(q * L, L)] for q in range(NCHUNK)]
        bf_c = [base_v[pl.ds(q * L, L)] for q in range(NCHUNK)]

        def chunk_terms(q, rn, lnf):
            t = lpf_c[q] - lnf
            pen = jnp.abs(t) * inv
            v = jnp.maximum(rp_c[q] - rn + pen, 0.0)
            vm = jnp.where(t != 0.0, v, 0.0)
            cm = jnp.where(t != 0.0, bf_c[q], 0.0)
            return vm * bf_c[q], cm

        def p2_body(n, carry):
            idx = jnp.full((L,), n, jnp.int32)
            rn = plsc.load_gather(r_v, [idx])
            lnf = plsc.load_gather(labf_v, [idx])
            out = list(carry)
            for q in range(NCHUNK):
                dv, dc = chunk_terms(q, rn, lnf)
                out[q] = carry[q] + dv
                out[NCHUNK + q] = carry[NCHUNK + q] + dc
            return tuple(out)

        sums = lax.fori_loop(0, N, p2_body, (zf,) * (2 * NCHUNK))

        idx_a = jnp.full((L,), a, jnp.int32)
        ra = plsc.load_gather(r_v, [idx_a])
        laf = plsc.load_gather(labf_v, [idx_a])
        for q in range(NCHUNK):
            dv, dc = chunk_terms(q, ra, laf)
            acc_tot = acc_tot + (sums[q] - dv)
            cnt_tot = cnt_tot + (sums[NCHUNK + q] - dc)

    part_v[0, pl.ds(0, L)] = acc_tot
    part_v[0, pl.ds(L, L)] = cnt_tot
    pltpu.sync_copy(part_v, shared.at[zidx_v], add=True)
    plsc.subcore_barrier()

    @pl.when(s == 0)
    def _():
        pltpu.sync_copy(shared, red_v)
        tsum = jnp.sum(red_v[0, pl.ds(0, L)])
        csum = jnp.sum(red_v[0, pl.ds(L, L)])
        wr_v[0, pl.ds(0, L)] = jnp.where(
            iota == 0, tsum, jnp.where(iota == 1, csum, 0.0))
        pltpu.sync_copy(wr_v, out_hbm.at[pl.ds(c, 1)])


def kernel(embeddings, target, max_score):
    et = embeddings.T.reshape(-1)
    maxf = jnp.broadcast_to(
        jnp.asarray(max_score).astype(jnp.float32), (L,))
    zidx = jnp.zeros((1,), jnp.int32)
    out = _triplet_sc(et, target, maxf, zidx)
    total = out[0, 0] + out[1, 0]
    cf = out[0, 1] + out[1, 1]
    mean = total / cf
    count = cf.astype(jnp.int32)
    return (mean, count)


# speedup vs baseline: 1.0561x; 1.0561x over previous
import functools

import jax
import jax.numpy as jnp
from jax import lax
from jax.experimental import pallas as pl
from jax.experimental.pallas import tpu as pltpu
from jax.experimental.pallas import tpu_sc as plsc

N = 64
D = 128
L = 16
NCHUNK = N // L
NC = 2
NS = 16
NW = NC * NS
APW = N // NW
NEG = -1e30

_mesh = plsc.VectorSubcoreMesh(
    core_axis_name="c", subcore_axis_name="s", num_cores=NC, num_subcores=NS
)


@functools.partial(
    pl.kernel,
    out_type=jax.ShapeDtypeStruct((NC, L), jnp.float32),
    mesh=_mesh,
    compiler_params=pltpu.CompilerParams(needs_layout_passes=False),
    scratch_types=[
        pltpu.VMEM((D * N,), jnp.float32),
        pltpu.VMEM((N,), jnp.int32),
        pltpu.VMEM((N,), jnp.float32),
        pltpu.VMEM((L,), jnp.float32),
        pltpu.VMEM((APW * N,), jnp.float32),
        pltpu.VMEM((L,), jnp.float32),
        pltpu.VMEM((1, 2 * L), jnp.float32),
        pltpu.VMEM((1, 2 * L), jnp.float32),
        pltpu.VMEM((1, L), jnp.float32),
        pltpu.VMEM((1,), jnp.int32),
        pltpu.VMEM_SHARED((1, 2 * L), jnp.float32),
        pltpu.SemaphoreType.DMA,
        pltpu.SemaphoreType.DMA,
    ],
)
def _triplet_sc(et_hbm, lab_hbm, max_hbm, zidx_hbm, out_hbm,
                et_v, lab_v, labf_v, histf_v, r_v, max_v,
                part_v, red_v, wr_v, zidx_v, shared, sem_s, sem_b):
    c = lax.axis_index("c")
    s = lax.axis_index("s")
    wid = s * NC + c
    a0 = wid * APW
    a1 = a0 + 1

    h_et = pltpu.async_copy(et_hbm, et_v, sem_b)
    h_lab = pltpu.async_copy(lab_hbm, lab_v, sem_s)
    h_max = pltpu.async_copy(max_hbm, max_v, sem_s)
    h_zi = pltpu.async_copy(zidx_hbm, zidx_v, sem_s)
    h_lab.wait()
    h_max.wait()
    h_zi.wait()

    iota = lax.iota(jnp.int32, L)
    zf = jnp.zeros((L,), jnp.float32)
    inv = 1.0 / max_v[...]

    @pl.when(s == 0)
    def _():
        red_v[0, pl.ds(0, L)] = zf
        red_v[0, pl.ds(L, L)] = zf
        pltpu.sync_copy(red_v, shared)

    for q in range(NCHUNK):
        lc = lab_v[pl.ds(q * L, L)]
        labf_v[pl.ds(q * L, L)] = lc.astype(jnp.float32)
    h = jnp.zeros((L,), jnp.int32)
    for j in range(N):
        lj = plsc.load_gather(lab_v, [jnp.full((L,), j, jnp.int32)])
        h = h + jnp.where(iota == lj, 1, 0).astype(jnp.int32)
    histf_v[...] = h.astype(jnp.float32)

    h_et.wait()

    def p1_body(d, accs):
        base = d * N
        ea0 = plsc.load_gather(et_v, [jnp.full((L,), base + a0, jnp.int32)])
        ea1 = plsc.load_gather(et_v, [jnp.full((L,), base + a1, jnp.int32)])
        out = []
        for q in range(NCHUNK):
            col = et_v[pl.ds(base + q * L, L)]
            d0 = col - ea0
            d1 = col - ea1
            out.append(accs[q] + d0 * d0)
            out.append(accs[NCHUNK + q] + d1 * d1)
        return tuple(out[0::2]) + tuple(out[1::2])

    accs = lax.fori_loop(0, D, p1_body, (zf,) * (2 * NCHUNK))
    for q in range(NCHUNK):
        r_v[pl.ds(q * L, L)] = accs[q]
        r_v[pl.ds(N + q * L, L)] = accs[NCHUNK + q]

    lpf_c = [labf_v[pl.ds(q * L, L)] for q in range(NCHUNK)]
    lab_c = [lab_v[pl.ds(q * L, L)] for q in range(NCHUNK)]
    cdf_c = [N - plsc.load_gather(histf_v, [lab_c[q]]) for q in range(NCHUNK)]
    laf0 = plsc.load_gather(labf_v, [jnp.full((L,), a0, jnp.int32)])
    laf1 = plsc.load_gather(labf_v, [jnp.full((L,), a1, jnp.int32)])

    rp_eff0 = []
    rp_eff1 = []
    bfs_c = []
    cnt_corr = zf
    for q in range(NCHUNK):
        pc = iota + q * L
        valid = cdf_c[q] >= 2.0
        cond0 = valid & (pc != a0)
        cond1 = valid & (pc != a1)
        bf0 = jnp.where(cond0, 1.0, 0.0)
        bf1 = jnp.where(cond1, 1.0, 0.0)
        bfs_c.append(bf0 + bf1)
        rp0 = r_v[pl.ds(q * L, L)]
        rp1 = r_v[pl.ds(N + q * L, L)]
        rp_eff0.append(jnp.where(cond0, rp0, NEG))
        rp_eff1.append(jnp.where(cond1, rp1, NEG))
        t0 = lpf_c[q] - laf0
        t1 = lpf_c[q] - laf1
        cnt_corr = (cnt_corr - jnp.where(t0 == 0.0, 0.0, bf0)
                    - jnp.where(t1 == 0.0, 0.0, bf1))

    def chunk_terms(q, rn0, rn1, lnf):
        t = lpf_c[q] - lnf
        pen = jnp.abs(t) * inv
        pen_eff = jnp.where(t == 0.0, NEG, pen)
        v0 = jnp.maximum(rp_eff0[q] - rn0 + pen_eff, 0.0)
        v1 = jnp.maximum(rp_eff1[q] - rn1 + pen_eff, 0.0)
        cm = jnp.where(t == 0.0, 0.0, bfs_c[q])
        return v0, v1, cm

    def p2_body(n, carry):
        idx = jnp.full((L,), n, jnp.int32)
        rn0 = plsc.load_gather(r_v, [idx])
        rn1 = plsc.load_gather(r_v, [idx + N])
        lnf = plsc.load_gather(labf_v, [idx])
        out = list(carry)
        for q in range(NCHUNK):
            v0, v1, cm = chunk_terms(q, rn0, rn1, lnf)
            out[q] = carry[q] + v0
            out[NCHUNK + q] = carry[NCHUNK + q] + v1
            out[2 * NCHUNK + q] = carry[2 * NCHUNK + q] + cm
        return tuple(out)

    sums = lax.fori_loop(0, N, p2_body, (zf,) * (3 * NCHUNK))

    ia0 = jnp.full((L,), a0, jnp.int32)
    ia1 = jnp.full((L,), a1, jnp.int32)
    ra00 = plsc.load_gather(r_v, [ia0])
    ra01 = plsc.load_gather(r_v, [ia0 + N])
    ra10 = plsc.load_gather(r_v, [ia1])
    ra11 = plsc.load_gather(r_v, [ia1 + N])
    acc_tot = zf
    cnt_tot = cnt_corr
    for q in range(NCHUNK):
        v0a, _, _ = chunk_terms(q, ra00, ra01, laf0)
        _, v1a, _ = chunk_terms(q, ra10, ra11, laf1)
        acc_tot = acc_tot + (sums[q] - v0a) + (sums[NCHUNK + q] - v1a)
        cnt_tot = cnt_tot + sums[2 * NCHUNK + q]

    part_v[0, pl.ds(0, L)] = acc_tot
    part_v[0, pl.ds(L, L)] = cnt_tot
    plsc.subcore_barrier()
    pltpu.sync_copy(part_v, shared.at[zidx_v], add=True)
    plsc.subcore_barrier()

    @pl.when(s == 0)
    def _():
        pltpu.sync_copy(shared, red_v)
        tsum = jnp.sum(red_v[0, pl.ds(0, L)])
        csum = jnp.sum(red_v[0, pl.ds(L, L)])
        wr_v[0, pl.ds(0, L)] = jnp.where(
            iota == 0, tsum, jnp.where(iota == 1, csum, 0.0))
        pltpu.sync_copy(wr_v, out_hbm.at[pl.ds(c, 1)])


def kernel(embeddings, target, max_score):
    et = embeddings.T.reshape(-1)
    maxf = jnp.broadcast_to(
        jnp.asarray(max_score).astype(jnp.float32), (L,))
    zidx = jnp.zeros((1,), jnp.int32)
    out = _triplet_sc(et, target, maxf, zidx)
    total = out[0, 0] + out[1, 0]
    cf = out[0, 1] + out[1, 1]
    mean = total / cf
    count = cf.astype(jnp.int32)
    return (mean, count)
